# baseline (speedup 1.0000x reference)
import functools

import jax
import jax.numpy as jnp
from jax import lax
from jax.experimental import pallas as pl
from jax.experimental.pallas import tpu as pltpu
from jax.experimental.pallas import tpu_sc as plsc

L = 16
NC = 2
NS = 16
NW = NC * NS
BATCH = 16384
D = 32
B_PER_W = BATCH // NW


def _mf_body(uids_hbm, mids_hbm, utab_hbm, mtab_hbm, out_hbm,
             uidx_v, midx_v, urows_v, mrows_v, outb_v, usem, msem):
    wid = lax.axis_index("s") * NC + lax.axis_index("c")
    base = wid * B_PER_W

    pltpu.sync_copy(uids_hbm.at[pl.ds(base, B_PER_W)], uidx_v)
    pltpu.sync_copy(mids_hbm.at[pl.ds(base, B_PER_W)], midx_v)

    copies = []
    for d in range(D):
        copies.append(pltpu.async_copy(
            utab_hbm.at[d].at[uidx_v], urows_v.at[d], usem))
        copies.append(pltpu.async_copy(
            mtab_hbm.at[d].at[midx_v], mrows_v.at[d], msem))
    for c in copies:
        c.wait()

    def body(g, carry):
        b0 = g * L
        acc = jnp.zeros((L,), jnp.float32)
        for d in range(D):
            u = urows_v[d, pl.ds(b0, L)]
            m = mrows_v[d, pl.ds(b0, L)]
            acc = acc + u * m
        outb_v[pl.ds(b0, L)] = acc
        return carry

    lax.fori_loop(0, B_PER_W // L, body, 0)
    pltpu.sync_copy(outb_v, out_hbm.at[pl.ds(base, B_PER_W)])


def kernel(user_ids, movie_ids, user_table, movie_table):
    utabT = user_table.T
    mtabT = movie_table.T
    mesh = plsc.VectorSubcoreMesh(core_axis_name="c", subcore_axis_name="s")
    f = functools.partial(
        pl.kernel,
        mesh=mesh,
        out_type=jax.ShapeDtypeStruct((BATCH,), jnp.float32),
        scratch_types=[
            pltpu.VMEM((B_PER_W,), jnp.int32),
            pltpu.VMEM((B_PER_W,), jnp.int32),
            pltpu.VMEM((D, B_PER_W), jnp.float32),
            pltpu.VMEM((D, B_PER_W), jnp.float32),
            pltpu.VMEM((B_PER_W,), jnp.float32),
            pltpu.SemaphoreType.DMA,
            pltpu.SemaphoreType.DMA,
        ],
        compiler_params=pltpu.CompilerParams(
            use_tc_tiling_on_sc=False, needs_layout_passes=False),
    )(_mf_body)
    return f(user_ids.astype(jnp.int32), movie_ids.astype(jnp.int32),
             utabT, mtabT)


# speedup vs baseline: 21.0251x; 21.0251x over previous
import functools

import jax
import jax.numpy as jnp
from jax import lax
from jax.experimental import pallas as pl
from jax.experimental.pallas import tpu as pltpu
from jax.experimental.pallas import tpu_sc as plsc

L = 16
NC = 2
NS = 16
NW = NC * NS
BATCH = 16384
D = 32
B_PER_W = BATCH // NW


def _mf_body(uids_hbm, mids_hbm, utab_hbm, mtab_hbm, out_hbm,
             uidx_v, midx_v, blk_v, comp_v, outb_v, usem):
    wid = lax.axis_index("s") * NC + lax.axis_index("c")
    base = wid * B_PER_W

    pltpu.sync_copy(uids_hbm.at[pl.ds(base, B_PER_W)], uidx_v)
    pltpu.sync_copy(mids_hbm.at[pl.ds(base, B_PER_W)], midx_v)

    lanes = lax.iota(jnp.int32, L)

    def fetch(tab, vec):
        lb = vec & ~127
        copies = []
        for j in range(L):
            lbj = pl.multiple_of(lb[j], 128)
            copies.append(pltpu.async_copy(
                tab.at[:, :, pl.ds(lbj, 128)], blk_v.at[j], usem))
        for cp in copies:
            cp.wait()

    def grp_body(g, carry):
        g0 = g * L
        uvec = uidx_v[pl.ds(g0, L)]
        mvec = midx_v[pl.ds(g0, L)]

        fetch(utab_hbm, uvec)
        uoff = uvec & 127
        for a in range(4):
            av = jnp.full((L,), a, jnp.int32)
            for s in range(8):
                sv = jnp.full((L,), s, jnp.int32)
                u = plsc.load_gather(blk_v, [lanes, av, sv, uoff])
                comp_v[a * 8 + s, pl.ds(0, L)] = u

        fetch(mtab_hbm, mvec)
        moff = mvec & 127
        acc = jnp.zeros((L,), jnp.float32)
        for a in range(4):
            av = jnp.full((L,), a, jnp.int32)
            for s in range(8):
                sv = jnp.full((L,), s, jnp.int32)
                m = plsc.load_gather(blk_v, [lanes, av, sv, moff])
                acc = acc + comp_v[a * 8 + s, pl.ds(0, L)] * m
        outb_v[pl.ds(g0, L)] = acc
        return carry

    lax.fori_loop(0, B_PER_W // L, grp_body, 0)
    pltpu.sync_copy(outb_v, out_hbm.at[pl.ds(base, B_PER_W)])


def kernel(user_ids, movie_ids, user_table, movie_table):
    n_users = user_table.shape[0]
    n_movies = movie_table.shape[0]
    utab3 = user_table.T.reshape(4, 8, n_users)
    mtab3 = movie_table.T.reshape(4, 8, n_movies)
    mesh = plsc.VectorSubcoreMesh(core_axis_name="c", subcore_axis_name="s")
    f = functools.partial(
        pl.kernel,
        mesh=mesh,
        out_type=jax.ShapeDtypeStruct((BATCH,), jnp.float32),
        scratch_types=[
            pltpu.VMEM((B_PER_W,), jnp.int32),
            pltpu.VMEM((B_PER_W,), jnp.int32),
            pltpu.VMEM((L, 4, 8, 128), jnp.float32),
            pltpu.VMEM((D, L), jnp.float32),
            pltpu.VMEM((B_PER_W,), jnp.float32),
            pltpu.SemaphoreType.DMA,
        ],
        compiler_params=pltpu.CompilerParams(
            use_tc_tiling_on_sc=True, needs_layout_passes=False),
    )(_mf_body)
    return f(user_ids.astype(jnp.int32), movie_ids.astype(jnp.int32),
             utab3, mtab3)
